# Initial kernel scaffold; baseline (speedup 1.0000x reference)
import jax
import jax.numpy as jnp
from jax.experimental import pallas as pl


def kernel(img):
    raise NotImplementedError("write your pallas kernel here")


# speedup vs baseline: 76.6595x; 76.6595x over previous
import functools

import jax
import jax.numpy as jnp
import numpy as np
from jax import lax
from jax.experimental import pallas as pl
from jax.experimental.pallas import tpu as pltpu
from jax.experimental.pallas import tpu_sc as plsc

NUM_R = 384
NUM_T = 180
W = 384
H = 384

NC = 2
NS = 16
LANES = 16
NW = NC * NS
ROWS_PER_W = W // NW
PIX_PER_W = ROWS_PER_W * H
NG = 12
T_PAD = NG * LANES
NWORDS = H // 32
ACC = NUM_T * NUM_R
RED = ACC // NS


def _build_tables():
    thetas = jnp.linspace(0.0, np.pi, NUM_T, endpoint=False)
    cos_t = jnp.cos(thetas)
    sin_t = jnp.sin(thetas)
    xs = jnp.arange(W, dtype=jnp.float32)
    ys = jnp.arange(H, dtype=jnp.float32)
    X, Y = jnp.meshgrid(xs, ys, indexing="ij")
    r = X[:, :, None] * cos_t[None, None, :] + Y[:, :, None] * sin_t[None, None, :]
    diag = float(np.sqrt(W * W + H * H))
    r_idx = jnp.clip(((r + diag) / (2.0 * diag) * NUM_R).astype(jnp.int32), 0, NUM_R - 1)

    bin0 = r_idx[:, 0, :]
    delta = r_idx[:, 1:, :] - r_idx[:, :-1, :]
    bits = jnp.concatenate(
        [jnp.zeros((W, 1, NUM_T), jnp.int32), delta], axis=1)
    shifts = jnp.arange(32, dtype=jnp.int32)
    bm = jnp.sum(bits.reshape(W, NWORDS, 32, NUM_T) << shifts[None, None, :, None],
                 axis=2).astype(jnp.int32)
    bm = jnp.pad(bm, ((0, 0), (0, 0), (0, T_PAD - NUM_T)))
    bin0 = jnp.pad(bin0, ((0, 0), (0, T_PAD - NUM_T)))
    return bm.reshape(-1), bin0.reshape(-1)


def _dht_body(img_hbm, bm_hbm, bin0_hbm, partials_hbm, out_hbm,
              img_v, bm_v, bin0_v, acc_v, tmp_v, red_v):
    c = lax.axis_index("c")
    s = lax.axis_index("s")
    wid = c * NS + s

    pltpu.sync_copy(img_hbm.at[pl.ds(wid * PIX_PER_W, PIX_PER_W)], img_v)
    pltpu.sync_copy(bm_hbm.at[pl.ds(wid * ROWS_PER_W * NWORDS * T_PAD,
                                    ROWS_PER_W * NWORDS * T_PAD)], bm_v)
    pltpu.sync_copy(bin0_hbm.at[pl.ds(wid * ROWS_PER_W * T_PAD,
                                      ROWS_PER_W * T_PAD)], bin0_v)

    zero16 = jnp.zeros((LANES,), jnp.float32)

    def zero_body(i, carry):
        acc_v[pl.ds(i * LANES, LANES)] = zero16
        return carry

    lax.fori_loop(0, ACC // LANES, zero_body, 0)

    lane = lax.iota(jnp.int32, LANES)
    tbases = [jnp.minimum(j * LANES + lane, NUM_T - 1) * NUM_R for j in range(NG)]
    masks = [None if (j + 1) * LANES <= NUM_T else (j * LANES + lane) < NUM_T
             for j in range(NG)]

    def x_body(xi, carry):
        def w_body(wi, flats):
            words = [bm_v[pl.ds((xi * NWORDS + wi) * T_PAD + j * LANES, LANES)]
                     for j in range(NG)]

            def y_body(y2, flats):
                yshift = jnp.full((LANES,), y2, jnp.int32)
                p = xi * H + wi * 32 + y2
                val = plsc.load_gather(img_v, [jnp.full((LANES,), p, jnp.int32)])
                new_flats = []
                for j in range(NG):
                    bit = (words[j] >> yshift) & 1
                    fl = flats[j] + bit
                    if masks[j] is None:
                        plsc.addupdate_scatter(acc_v, [fl], val)
                    else:
                        plsc.addupdate_scatter(acc_v, [fl], val, mask=masks[j])
                    new_flats.append(fl)
                return tuple(new_flats)

            return lax.fori_loop(0, 32, y_body, flats)

        flats0 = tuple(
            bin0_v[pl.ds(xi * T_PAD + j * LANES, LANES)] + tbases[j]
            for j in range(NG))
        lax.fori_loop(0, NWORDS, w_body, flats0)
        return carry

    lax.fori_loop(0, ROWS_PER_W, x_body, 0)

    pltpu.sync_copy(acc_v, partials_hbm.at[pl.ds(wid * ACC, ACC)])
    plsc.subcore_barrier()

    pltpu.sync_copy(partials_hbm.at[pl.ds(c * NS * ACC + s * RED, RED)], red_v)

    def add_body(i, carry):
        sl = pl.ds(i * LANES, LANES)
        red_v[sl] = red_v[sl] + tmp_v[sl]
        return carry

    for k in range(1, NS):
        pltpu.sync_copy(
            partials_hbm.at[pl.ds((c * NS + k) * ACC + s * RED, RED)], tmp_v)
        lax.fori_loop(0, RED // LANES, add_body, 0)

    pltpu.sync_copy(red_v, out_hbm.at[pl.ds(c * ACC + s * RED, RED)])


@jax.jit
def kernel(img):
    bm, bin0 = _build_tables()
    mesh = plsc.VectorSubcoreMesh(core_axis_name="c", subcore_axis_name="s")
    call = functools.partial(
        pl.kernel,
        out_type=(
            jax.ShapeDtypeStruct((NC * NS * ACC,), jnp.float32),
            jax.ShapeDtypeStruct((NC * ACC,), jnp.float32),
        ),
        mesh=mesh,
        compiler_params=pltpu.CompilerParams(needs_layout_passes=False),
        scratch_types=[
            pltpu.VMEM((PIX_PER_W,), jnp.float32),
            pltpu.VMEM((ROWS_PER_W * NWORDS * T_PAD,), jnp.int32),
            pltpu.VMEM((ROWS_PER_W * T_PAD,), jnp.int32),
            pltpu.VMEM((ACC,), jnp.float32),
            pltpu.VMEM((RED,), jnp.float32),
            pltpu.VMEM((RED,), jnp.float32),
        ],
    )
    _, out2 = call(_dht_body)(img.reshape(-1), bm, bin0)
    return out2.reshape(NC, NUM_T, NUM_R).sum(axis=0).T
